# Initial kernel scaffold; baseline (speedup 1.0000x reference)
import jax
import jax.numpy as jnp
from jax.experimental import pallas as pl


def kernel(feats, edge_index, etype, W1, Wloop1, b1, W2, Wloop2, b2):
    raise NotImplementedError("write your pallas kernel here")


# speedup vs baseline: 2.7845x; 2.7845x over previous
import functools

import jax
import jax.numpy as jnp
from jax import lax
from jax.experimental import pallas as pl
from jax.experimental.pallas import tpu as pltpu
from jax.experimental.pallas import tpu_sc as plsc

NC = 2
NS = 16
NW = NC * NS
K = 128
GC = 16
LANES = 16


def _transform_body(x_ref, w_ref, y_ref):
    y_ref[0] = jnp.dot(x_ref[:], w_ref[0],
                       preferred_element_type=jnp.float32)


def _tc_transform(x, W, bn):
    n, d = x.shape
    r = W.shape[0]
    return pl.pallas_call(
        _transform_body,
        grid=(n // bn, r),
        in_specs=[
            pl.BlockSpec((bn, d), lambda i, j: (i, 0)),
            pl.BlockSpec((1, d, d), lambda i, j: (j, 0, 0)),
        ],
        out_specs=pl.BlockSpec((1, bn, d), lambda i, j: (j, i, 0)),
        out_shape=jax.ShapeDtypeStruct((r, n, d), jnp.float32),
    )(x, W)


def _combine_body(p_ref, x_ref, wl_ref, b_ref, o_ref, *, act):
    h = (p_ref[0] + p_ref[1]
         + jnp.dot(x_ref[:], wl_ref[:], preferred_element_type=jnp.float32)
         + b_ref[:])
    o_ref[:] = jnp.maximum(h, 0.0) if act else h


def _tc_combine(parts, x, Wloop, b, act, bn):
    n, d = x.shape
    return pl.pallas_call(
        functools.partial(_combine_body, act=act),
        grid=(n // bn,),
        in_specs=[
            pl.BlockSpec((2, bn, d), lambda i: (0, i, 0)),
            pl.BlockSpec((bn, d), lambda i: (i, 0)),
            pl.BlockSpec((d, d), lambda i: (0, 0)),
            pl.BlockSpec((1, d), lambda i: (0, 0)),
        ],
        out_specs=pl.BlockSpec((bn, d), lambda i: (i, 0)),
        out_shape=jax.ShapeDtypeStruct((n, d), jnp.float32),
    )(parts, x, Wloop, b.reshape(1, d))


def _sc_agg_body(tbl, srcw, etw, dstw, out, src_v, et_v, dst_v,
                 rows0, rows1, acc, sem0, sem1, *, G, n, n_pad, D):
    c = lax.axis_index("c")
    s = lax.axis_index("s")
    wid = s * NC + c
    rows_per_tile = n_pad // NS

    zv = jnp.zeros((LANES,), jnp.float32)
    def _z(i, carry):
        src_row = i // (D // LANES)
        lane0 = (i % (D // LANES)) * LANES
        rows0[src_row, pl.ds(lane0, LANES)] = zv
        return carry
    lax.fori_loop(0, K * D // LANES, _z, 0)
    for t in range(rows_per_tile // K):
        pltpu.sync_copy(rows0, acc.at[pl.ds(s * rows_per_tile + t * K, K)])
    plsc.subcore_barrier()

    def _start(g, buf, sem):
        pltpu.async_copy(tbl.at[src_v.at[g]], buf, sem)

    def _wait(buf, sem):
        pltpu.make_async_copy(tbl.at[src_v.at[0]], buf, sem).wait()

    def _add(g, buf):
        pltpu.sync_copy(buf, acc.at[dst_v.at[g]], add=True)

    G_SUPER = G // GC
    def _super(si, carry):
        pltpu.sync_copy(srcw.at[wid, pl.ds(si * GC, GC)], src_v)
        pltpu.sync_copy(etw.at[wid, pl.ds(si * GC, GC)], et_v)
        pltpu.sync_copy(dstw.at[wid, pl.ds(si * GC, GC)], dst_v)

        def _gix(g, c2):
            for j in range(K // LANES):
                sl = pl.ds(j * LANES, LANES)
                src_v[g, sl] = et_v[g, sl] * n + src_v[g, sl]
            return c2
        lax.fori_loop(0, GC, _gix, 0)

        _start(0, rows0, sem0)
        def _step(i, c2):
            g = 2 * i
            _start(g + 1, rows1, sem1)
            _wait(rows0, sem0)
            _add(g, rows0)
            _start(g + 2, rows0, sem0)
            _wait(rows1, sem1)
            _add(g + 1, rows1)
            return c2
        lax.fori_loop(0, (GC - 2) // 2, _step, 0)
        _start(GC - 1, rows1, sem1)
        _wait(rows0, sem0)
        _add(GC - 2, rows0)
        _wait(rows1, sem1)
        _add(GC - 1, rows1)
        return carry
    lax.fori_loop(0, G_SUPER, _super, 0)

    plsc.subcore_barrier()

    row0 = s * rows_per_tile
    pltpu.sync_copy(acc.at[pl.ds(row0, rows_per_tile)],
                    out.at[c, pl.ds(row0, rows_per_tile)])


def _sc_agg(table, src_r, et_r, dst_r, G, n, n_pad, D):
    mesh = plsc.VectorSubcoreMesh(core_axis_name="c", subcore_axis_name="s",
                                  num_cores=NC, num_subcores=NS)
    body = functools.partial(_sc_agg_body, G=G, n=n, n_pad=n_pad, D=D)
    return pl.kernel(
        body,
        out_type=jax.ShapeDtypeStruct((NC, n_pad, D), jnp.float32),
        mesh=mesh,
        scratch_types=[
            pltpu.VMEM((GC, K), jnp.int32),
            pltpu.VMEM((GC, K), jnp.int32),
            pltpu.VMEM((GC, K), jnp.int32),
            pltpu.VMEM((K, D), jnp.float32),
            pltpu.VMEM((K, D), jnp.float32),
            pltpu.VMEM_SHARED((n_pad, D), jnp.float32),
            pltpu.SemaphoreType.DMA,
            pltpu.SemaphoreType.DMA,
        ],
    )(table, src_r, et_r, dst_r)


def kernel(feats, edge_index, etype, W1, Wloop1, b1, W2, Wloop2, b2):
    n, d = feats.shape
    r = W1.shape[0]
    e = etype.shape[0]

    per_w = -(-e // (NW * GC * K)) * (GC * K)
    G = per_w // K
    e_pad = NW * per_w
    n_pad = -(-n // (NS * K)) * (NS * K)

    src = jnp.concatenate(
        [edge_index[0], jnp.zeros((e_pad - e,), jnp.int32)]).reshape(NW, G, K)
    et = jnp.concatenate(
        [etype, jnp.zeros((e_pad - e,), jnp.int32)]).reshape(NW, G, K)
    dst = jnp.concatenate(
        [edge_index[1], jnp.full((e_pad - e,), n, jnp.int32)]).reshape(NW, G, K)

    bn = 1000
    h = feats
    for (W, Wloop, b, act) in ((W1, Wloop1, b1, True),
                               (W2, Wloop2, b2, False)):
        table = _tc_transform(h, W, bn).reshape(n * r, d)
        parts = _sc_agg(table, src, et, dst, G, n, n_pad, d)
        h = _tc_combine(parts, h, Wloop, b, act, bn)
    return h


# speedup vs baseline: 2.8208x; 1.0130x over previous
import functools

import jax
import jax.numpy as jnp
from jax import lax
from jax.experimental import pallas as pl
from jax.experimental.pallas import tpu as pltpu
from jax.experimental.pallas import tpu_sc as plsc

NC = 2
NS = 16
NW = NC * NS
K = 80
GC = 16
NB = 4
LANES = 16


def _transform_body(x_ref, w_ref, y_ref):
    y_ref[0] = jnp.dot(x_ref[:], w_ref[0],
                       preferred_element_type=jnp.float32)


def _tc_transform(x, W, bn):
    n, d = x.shape
    r = W.shape[0]
    return pl.pallas_call(
        _transform_body,
        grid=(n // bn, r),
        in_specs=[
            pl.BlockSpec((bn, d), lambda i, j: (i, 0)),
            pl.BlockSpec((1, d, d), lambda i, j: (j, 0, 0)),
        ],
        out_specs=pl.BlockSpec((1, bn, d), lambda i, j: (j, i, 0)),
        out_shape=jax.ShapeDtypeStruct((r, n, d), jnp.float32),
    )(x, W)


def _combine_body(p_ref, x_ref, wl_ref, b_ref, o_ref, *, act):
    h = (p_ref[0] + p_ref[1]
         + jnp.dot(x_ref[:], wl_ref[:], preferred_element_type=jnp.float32)
         + b_ref[:])
    o_ref[:] = jnp.maximum(h, 0.0) if act else h


def _tc_combine(parts, x, Wloop, b, act, bn):
    n, d = x.shape
    return pl.pallas_call(
        functools.partial(_combine_body, act=act),
        grid=(n // bn,),
        in_specs=[
            pl.BlockSpec((2, bn, d), lambda i: (0, i, 0)),
            pl.BlockSpec((bn, d), lambda i: (i, 0)),
            pl.BlockSpec((d, d), lambda i: (0, 0)),
            pl.BlockSpec((1, d), lambda i: (0, 0)),
        ],
        out_specs=pl.BlockSpec((bn, d), lambda i: (i, 0)),
        out_shape=jax.ShapeDtypeStruct((n, d), jnp.float32),
    )(parts, x, Wloop, b.reshape(1, d))


def _sc_agg_body(tbl, srcw, etw, dstw, out, src_v, et_v, dst_v,
                 r0, r1, r2, r3, acc, gs0, gs1, gs2, gs3,
                 as0, as1, as2, as3, *, G, n, n_pad, D):
    bufs = (r0, r1, r2, r3)
    gsems = (gs0, gs1, gs2, gs3)
    asems = (as0, as1, as2, as3)
    c = lax.axis_index("c")
    s = lax.axis_index("s")
    wid = s * NC + c
    rows_per_tile = n_pad // NS

    zv = jnp.zeros((LANES,), jnp.float32)
    def _z(i, carry):
        row = i // (D // LANES)
        lane0 = (i % (D // LANES)) * LANES
        r0[row, pl.ds(lane0, LANES)] = zv
        return carry
    lax.fori_loop(0, K * D // LANES, _z, 0)
    for t in range(rows_per_tile // K):
        pltpu.sync_copy(r0, acc.at[pl.ds(s * rows_per_tile + t * K, K)])
    plsc.subcore_barrier()

    def _start(g, b):
        pltpu.async_copy(tbl.at[src_v.at[g]], bufs[b], gsems[b])

    def _wgather(b):
        pltpu.make_async_copy(tbl.at[src_v.at[0]], bufs[b], gsems[b]).wait()

    def _sadd(g, b):
        pltpu.async_copy(bufs[b], acc.at[dst_v.at[g]], asems[b], add=True)

    def _wadd(b):
        pltpu.make_async_copy(bufs[b], acc.at[dst_v.at[0]], asems[b]).wait()

    G_SUPER = G // GC
    def _super(si, carry):
        pltpu.sync_copy(srcw.at[wid, pl.ds(si * GC, GC)], src_v)
        pltpu.sync_copy(etw.at[wid, pl.ds(si * GC, GC)], et_v)
        pltpu.sync_copy(dstw.at[wid, pl.ds(si * GC, GC)], dst_v)

        def _gix(g, c2):
            for j in range(K // LANES):
                sl = pl.ds(j * LANES, LANES)
                src_v[g, sl] = et_v[g, sl] * n + src_v[g, sl]
            return c2
        lax.fori_loop(0, GC, _gix, 0)

        for b in range(NB):
            _start(b, b)
        for rd in range(GC // NB):
            base = rd * NB
            for b in range(NB):
                _wgather(b)
                _sadd(base + b, b)
            if rd < GC // NB - 1:
                for b in range(NB):
                    _wadd(b)
                    _start(base + NB + b, b)
            else:
                for b in range(NB):
                    _wadd(b)
        return carry
    lax.fori_loop(0, G_SUPER, _super, 0)

    plsc.subcore_barrier()

    row0 = s * rows_per_tile
    pltpu.sync_copy(acc.at[pl.ds(row0, rows_per_tile)],
                    out.at[c, pl.ds(row0, rows_per_tile)])


def _sc_agg(table, src_r, et_r, dst_r, G, n, n_pad, D):
    mesh = plsc.VectorSubcoreMesh(core_axis_name="c", subcore_axis_name="s",
                                  num_cores=NC, num_subcores=NS)
    body = functools.partial(_sc_agg_body, G=G, n=n, n_pad=n_pad, D=D)
    return pl.kernel(
        body,
        out_type=jax.ShapeDtypeStruct((NC, n_pad, D), jnp.float32),
        mesh=mesh,
        scratch_types=(
            [pltpu.VMEM((GC, K), jnp.int32)] * 3
            + [pltpu.VMEM((K, D), jnp.float32)] * NB
            + [pltpu.VMEM_SHARED((n_pad, D), jnp.float32)]
            + [pltpu.SemaphoreType.DMA] * (2 * NB)
        ),
    )(table, src_r, et_r, dst_r)


def kernel(feats, edge_index, etype, W1, Wloop1, b1, W2, Wloop2, b2):
    n, d = feats.shape
    r = W1.shape[0]
    e = etype.shape[0]

    per_w = -(-e // (NW * GC * K)) * (GC * K)
    G = per_w // K
    e_pad = NW * per_w
    n_pad = -(-n // (NS * K)) * (NS * K)

    src = jnp.concatenate(
        [edge_index[0], jnp.zeros((e_pad - e,), jnp.int32)]).reshape(NW, G, K)
    et = jnp.concatenate(
        [etype, jnp.zeros((e_pad - e,), jnp.int32)]).reshape(NW, G, K)
    dst = jnp.concatenate(
        [edge_index[1], jnp.full((e_pad - e,), n, jnp.int32)]).reshape(NW, G, K)

    bn = 1000
    h = feats
    for (W, Wloop, b, act) in ((W1, Wloop1, b1, True),
                               (W2, Wloop2, b2, False)):
        table = _tc_transform(h, W, bn).reshape(n * r, d)
        parts = _sc_agg(table, src, et, dst, G, n, n_pad, d)
        h = _tc_combine(parts, h, Wloop, b, act, bn)
    return h
